# baseline (speedup 1.0000x reference)
import functools

import jax
import jax.numpy as jnp
import numpy as np
from jax.experimental import pallas as pl
from jax.experimental.pallas import tpu as pltpu

N_NODES = 10000
N_GRAPHS = 8
K = 5
EPS = 1e-5

N_PAD = 10240
ROW_TILE = 256
N_ROW_TILES = N_PAD // ROW_TILE


def _stage3_body(lo_ref, hi_ref, x_ref, batch_ref, w5_ref, b5_ref, out_ref):
    i = pl.program_id(0)

    @pl.when(i == 0)
    def _init():
        out_ref[...] = jnp.full_like(out_ref, -jnp.inf)

    z = jax.lax.dot_general(x_ref[...], w5_ref[...], (((1,), (1,)), ((), ())),
                            preferred_element_type=jnp.float32)
    z = z + b5_ref[...]
    lo = lo_ref[i]
    hi = hi_ref[i]
    batch = batch_ref[...]

    def body(g, _):
        mask = batch == g
        zm = jnp.max(jnp.where(mask, z, -jnp.inf), axis=0, keepdims=True)
        cur = out_ref[pl.ds(g, 1), :]
        out_ref[pl.ds(g, 1), :] = jnp.maximum(cur, zm)
        return 0

    jax.lax.fori_loop(lo, hi + 1, body, 0)


def _stage3_pool(x, batch_pad, lo, hi, W5, b5):
    grid_spec = pltpu.PrefetchScalarGridSpec(
        num_scalar_prefetch=2,
        grid=(N_ROW_TILES,),
        in_specs=[
            pl.BlockSpec((ROW_TILE, 192), lambda i, lo, hi: (i, 0)),
            pl.BlockSpec((ROW_TILE, 1), lambda i, lo, hi: (i, 0)),
            pl.BlockSpec((1024, 192), lambda i, lo, hi: (0, 0)),
            pl.BlockSpec((1, 1024), lambda i, lo, hi: (0, 0)),
        ],
        out_specs=pl.BlockSpec((16, 1024), lambda i, lo, hi: (0, 0)),
    )
    out = pl.pallas_call(
        _stage3_body,
        grid_spec=grid_spec,
        out_shape=jax.ShapeDtypeStruct((16, 1024), jnp.float32),
    )(lo, hi, x, batch_pad, W5, b5.reshape(1, 1024))
    return out[:N_GRAPHS]


def _head_body(z_ref, s5_ref, be5_ref, w6_ref, b6_ref, s6_ref, be6_ref,
               w7_ref, b7_ref, s7_ref, be7_ref, w8_ref, b8_ref, out_ref):
    x = jnp.maximum(z_ref[...], 0.0) * s5_ref[...] + be5_ref[...]
    z = jax.lax.dot_general(x, w6_ref[...], (((1,), (1,)), ((), ())),
                            preferred_element_type=jnp.float32) + b6_ref[...]
    x = jnp.maximum(z, 0.0) * s6_ref[...] + be6_ref[...]
    z = jax.lax.dot_general(x, w7_ref[...], (((1,), (1,)), ((), ())),
                            preferred_element_type=jnp.float32) + b7_ref[...]
    x = jnp.maximum(z, 0.0) * s7_ref[...] + be7_ref[...]
    out_ref[...] = jax.lax.dot_general(x, w8_ref[...], (((1,), (1,)), ((), ())),
                                       preferred_element_type=jnp.float32) \
        + b8_ref[...]


def _head(z5max, p, s):
    args = (z5max, s['s5'].reshape(1, -1), p['be5'].reshape(1, -1),
            p['W6'], p['b6'].reshape(1, -1), s['s6'].reshape(1, -1),
            p['be6'].reshape(1, -1),
            p['W7'], p['b7'].reshape(1, -1), s['s7'].reshape(1, -1),
            p['be7'].reshape(1, -1),
            p['W8'], p['b8'].reshape(1, -1))
    return pl.pallas_call(
        _head_body,
        out_shape=jax.ShapeDtypeStruct((N_GRAPHS, 40), jnp.float32),
    )(*args)


KNN_RT = 128
KNN_CT = 512
KNN_NI = N_PAD // KNN_RT
KNN_NJ = N_PAD // KNN_CT
IMAX = np.int32(0x7FFFFFFF)


def _knn_body(rlo, rhi, clo, chi, xall_ref, batchr_ref, batchc_ref, out_ref,
              td, ti):
    i = pl.program_id(0)
    j = pl.program_id(1)

    @pl.when(j == 0)
    def _init():
        td[...] = jnp.full_like(td, jnp.inf)
        ti[...] = jnp.full_like(ti, IMAX)

    active = jnp.logical_and(rhi[i] >= clo[j], rlo[i] <= chi[j])

    @pl.when(active)
    def _merge():
        xr = xall_ref[pl.ds(i * KNN_RT, KNN_RT), :]
        xc = xall_ref[pl.ds(j * KNN_CT, KNN_CT), :]
        r = jax.lax.dot_general(xr, xc, (((1,), (1,)), ((), ())),
                                preferred_element_type=jnp.float32)
        sqc = jnp.sum(xc * xc, axis=1)
        r = sqc[None, :] - 2.0 * r
        bm = batchr_ref[...] != batchc_ref[...]
        r = jnp.where(bm, jnp.inf, r)
        colid = jax.lax.broadcasted_iota(jnp.int32, (KNN_RT, KNN_CT), 1) \
            + j * KNN_CT
        cd = jnp.concatenate([td[...], r], axis=1)
        ci = jnp.concatenate([ti[...], colid], axis=1)
        nd, ni = [], []
        for k in range(K):
            m = jnp.min(cd, axis=1, keepdims=True)
            sel = jnp.min(jnp.where(cd == m, ci, IMAX), axis=1, keepdims=True)
            nd.append(m)
            ni.append(sel)
            cd = jnp.where(ci == sel, jnp.inf, cd)
        td[...] = jnp.concatenate(
            nd + [jnp.full((KNN_RT, 8 - K), jnp.inf)], axis=1)
        ti[...] = jnp.concatenate(
            ni + [jnp.full((KNN_RT, 8 - K), IMAX)], axis=1)

    @pl.when(j == KNN_NJ - 1)
    def _emit():
        out_ref[...] = ti[...]


def _knn_topk(x1_pad, batch_pad):
    bt_r = batch_pad.reshape(KNN_NI, KNN_RT)
    bt_c = batch_pad.reshape(KNN_NJ, KNN_CT)
    rlo = bt_r[:, 0].astype(jnp.int32)
    rhi = bt_r[:, -1].astype(jnp.int32)
    clo = bt_c[:, 0].astype(jnp.int32)
    chi = bt_c[:, -1].astype(jnp.int32)
    grid_spec = pltpu.PrefetchScalarGridSpec(
        num_scalar_prefetch=4,
        grid=(KNN_NI, KNN_NJ),
        in_specs=[
            pl.BlockSpec((N_PAD, 64), lambda i, j, *_: (0, 0)),
            pl.BlockSpec((KNN_RT, 1), lambda i, j, *_: (i, 0)),
            pl.BlockSpec((1, KNN_CT), lambda i, j, *_: (0, j)),
        ],
        out_specs=pl.BlockSpec((KNN_RT, 8), lambda i, j, *_: (i, 0)),
        scratch_shapes=[pltpu.VMEM((KNN_RT, 8), jnp.float32),
                        pltpu.VMEM((KNN_RT, 8), jnp.int32)],
    )
    idx8 = pl.pallas_call(
        _knn_body,
        grid_spec=grid_spec,
        out_shape=jax.ShapeDtypeStruct((N_PAD, 8), jnp.int32),
        compiler_params=pltpu.CompilerParams(
            dimension_semantics=("arbitrary", "arbitrary")),
    )(rlo, rhi, clo, chi, x1_pad,
      batch_pad.reshape(N_PAD, 1), batch_pad.reshape(1, N_PAD))
    return idx8[:N_NODES, :K]


def _block(x, W, b, g, be):
    h = jnp.maximum(x @ W.T + b, 0.0)
    return h * (g / jnp.sqrt(1.0 + EPS)) + be


def kernel(pos, batch, edge_index, params):
    p = params
    n = pos.shape[0]
    scales = {k: p['g' + k[1]] / jnp.sqrt(1.0 + EPS)
              for k in ('s1', 's2', 's3', 's4', 's5', 's6', 's7')}

    loops = jnp.arange(n, dtype=edge_index.dtype)
    src = jnp.concatenate([edge_index[0], loops])
    dst = jnp.concatenate([edge_index[1], loops])
    x_i = pos[dst]
    x_j = pos[src]
    m = jnp.concatenate([x_i, x_j - x_i], axis=1)
    m = _block(m, p['W1'], p['b1'], p['g1'], p['be1'])
    m = _block(m, p['W2'], p['b2'], p['g2'], p['be2'])
    m = _block(m, p['W3'], p['b3'], p['g3'], p['be3'])
    x1 = jax.ops.segment_max(m, dst, num_segments=n)

    x1_pad = jnp.pad(x1, ((0, N_PAD - n), (0, 0)))
    batch_pad_1d = jnp.pad(batch.astype(jnp.int32), (0, N_PAD - n),
                           constant_values=N_GRAPHS)
    idx = _knn_topk(x1_pad, batch_pad_1d)
    xj = x1[idx]
    xi = x1[:, None, :]
    m2 = jnp.concatenate([jnp.broadcast_to(xi, xj.shape), xj - xi], axis=2)
    m2 = _block(m2, p['W4'], p['b4'], p['g4'], p['be4'])
    x2 = jnp.max(m2, axis=1)

    h = jnp.concatenate([x1, x2], axis=1)
    h = jnp.pad(h, ((0, N_PAD - n), (0, 0)))
    batch_pad = jnp.pad(batch.astype(jnp.int32), (0, N_PAD - n),
                        constant_values=N_GRAPHS).reshape(N_PAD, 1)
    bt = batch_pad.reshape(N_ROW_TILES, ROW_TILE)
    lo = jnp.min(bt, axis=1).astype(jnp.int32)
    hi = jnp.max(bt, axis=1).astype(jnp.int32)
    z5max = _stage3_pool(h, batch_pad, lo, hi, p['W5'], p['b5'])
    return _head(z5max, p, scales)


# speedup vs baseline: 2.6233x; 2.6233x over previous
import functools

import jax
import jax.numpy as jnp
import numpy as np
from jax.experimental import pallas as pl
from jax.experimental.pallas import tpu as pltpu

N_NODES = 10000
N_GRAPHS = 8
K = 5
EPS = 1e-5

N_PAD = 10240
ROW_TILE = 256
N_ROW_TILES = N_PAD // ROW_TILE


def _stage3_body(lo_ref, hi_ref, x_ref, batch_ref, w5_ref, b5_ref, out_ref):
    i = pl.program_id(0)

    @pl.when(i == 0)
    def _init():
        out_ref[...] = jnp.full_like(out_ref, -jnp.inf)

    z = jax.lax.dot_general(x_ref[...], w5_ref[...], (((1,), (1,)), ((), ())),
                            preferred_element_type=jnp.float32)
    z = z + b5_ref[...]
    lo = lo_ref[i]
    hi = hi_ref[i]
    batch = batch_ref[...]

    def body(g, _):
        mask = batch == g
        zm = jnp.max(jnp.where(mask, z, -jnp.inf), axis=0, keepdims=True)
        cur = out_ref[pl.ds(g, 1), :]
        out_ref[pl.ds(g, 1), :] = jnp.maximum(cur, zm)
        return 0

    jax.lax.fori_loop(lo, hi + 1, body, 0)


def _stage3_pool(x, batch_pad, lo, hi, W5, b5):
    grid_spec = pltpu.PrefetchScalarGridSpec(
        num_scalar_prefetch=2,
        grid=(N_ROW_TILES,),
        in_specs=[
            pl.BlockSpec((ROW_TILE, 192), lambda i, lo, hi: (i, 0)),
            pl.BlockSpec((ROW_TILE, 1), lambda i, lo, hi: (i, 0)),
            pl.BlockSpec((1024, 192), lambda i, lo, hi: (0, 0)),
            pl.BlockSpec((1, 1024), lambda i, lo, hi: (0, 0)),
        ],
        out_specs=pl.BlockSpec((16, 1024), lambda i, lo, hi: (0, 0)),
    )
    out = pl.pallas_call(
        _stage3_body,
        grid_spec=grid_spec,
        out_shape=jax.ShapeDtypeStruct((16, 1024), jnp.float32),
    )(lo, hi, x, batch_pad, W5, b5.reshape(1, 1024))
    return out[:N_GRAPHS]


def _head_body(z_ref, s5_ref, be5_ref, w6_ref, b6_ref, s6_ref, be6_ref,
               w7_ref, b7_ref, s7_ref, be7_ref, w8_ref, b8_ref, out_ref):
    x = jnp.maximum(z_ref[...], 0.0) * s5_ref[...] + be5_ref[...]
    z = jax.lax.dot_general(x, w6_ref[...], (((1,), (1,)), ((), ())),
                            preferred_element_type=jnp.float32) + b6_ref[...]
    x = jnp.maximum(z, 0.0) * s6_ref[...] + be6_ref[...]
    z = jax.lax.dot_general(x, w7_ref[...], (((1,), (1,)), ((), ())),
                            preferred_element_type=jnp.float32) + b7_ref[...]
    x = jnp.maximum(z, 0.0) * s7_ref[...] + be7_ref[...]
    out_ref[...] = jax.lax.dot_general(x, w8_ref[...], (((1,), (1,)), ((), ())),
                                       preferred_element_type=jnp.float32) \
        + b8_ref[...]


def _head(z5max, p, s):
    args = (z5max, s['s5'].reshape(1, -1), p['be5'].reshape(1, -1),
            p['W6'], p['b6'].reshape(1, -1), s['s6'].reshape(1, -1),
            p['be6'].reshape(1, -1),
            p['W7'], p['b7'].reshape(1, -1), s['s7'].reshape(1, -1),
            p['be7'].reshape(1, -1),
            p['W8'], p['b8'].reshape(1, -1))
    return pl.pallas_call(
        _head_body,
        out_shape=jax.ShapeDtypeStruct((N_GRAPHS, 40), jnp.float32),
    )(*args)


KNN_RT = 128
KNN_CT = 512
KNN_NI = N_PAD // KNN_RT
KNN_NJ = N_PAD // KNN_CT
IMAX = np.int32(0x7FFFFFFF)


def _knn_body(rlo, rhi, clo, chi, xall_ref, batchr_ref, batchc_ref, out_ref,
              td, ti):
    i = pl.program_id(0)
    j = pl.program_id(1)

    @pl.when(j == 0)
    def _init():
        td[...] = jnp.full_like(td, jnp.inf)
        ti[...] = jnp.full_like(ti, IMAX)

    active = jnp.logical_and(rhi[i] >= clo[j], rlo[i] <= chi[j])

    @pl.when(active)
    def _merge():
        xr = xall_ref[pl.ds(i * KNN_RT, KNN_RT), :]
        xc = xall_ref[pl.ds(j * KNN_CT, KNN_CT), :]
        r = jax.lax.dot_general(xr, xc, (((1,), (1,)), ((), ())),
                                preferred_element_type=jnp.float32)
        sqc = jax.lax.dot_general(jnp.ones((1, 64), jnp.float32), xc * xc,
                                  (((1,), (1,)), ((), ())),
                                  preferred_element_type=jnp.float32)
        r = sqc - 2.0 * r
        bm = batchr_ref[...] != batchc_ref[...]
        r = jnp.where(bm, jnp.inf, r)
        colid = jax.lax.broadcasted_iota(jnp.int32, (KNN_RT, KNN_CT), 1) \
            + j * KNN_CT
        cd = jnp.concatenate([td[...], r], axis=1)
        ci = jnp.concatenate([ti[...], colid], axis=1)
        for k in range(K):
            m = jnp.min(cd, axis=1, keepdims=True)
            sel = jnp.min(jnp.where(cd == m, ci, IMAX), axis=1, keepdims=True)
            td[:, k:k + 1] = m
            ti[:, k:k + 1] = sel
            cd = jnp.where(ci == sel, jnp.inf, cd)

    @pl.when(j == KNN_NJ - 1)
    def _emit():
        out_ref[...] = ti[:, :8]


def _knn_topk(x1_pad, batch_pad):
    bt_r = batch_pad.reshape(KNN_NI, KNN_RT)
    bt_c = batch_pad.reshape(KNN_NJ, KNN_CT)
    rlo = bt_r[:, 0].astype(jnp.int32)
    rhi = bt_r[:, -1].astype(jnp.int32)
    clo = bt_c[:, 0].astype(jnp.int32)
    chi = bt_c[:, -1].astype(jnp.int32)
    grid_spec = pltpu.PrefetchScalarGridSpec(
        num_scalar_prefetch=4,
        grid=(KNN_NI, KNN_NJ),
        in_specs=[
            pl.BlockSpec((N_PAD, 64), lambda i, j, *_: (0, 0)),
            pl.BlockSpec((KNN_RT, 1), lambda i, j, *_: (i, 0)),
            pl.BlockSpec((1, KNN_CT), lambda i, j, *_: (0, j)),
        ],
        out_specs=pl.BlockSpec((KNN_RT, 8), lambda i, j, *_: (i, 0)),
        scratch_shapes=[pltpu.VMEM((KNN_RT, 128), jnp.float32),
                        pltpu.VMEM((KNN_RT, 128), jnp.int32)],
    )
    idx8 = pl.pallas_call(
        _knn_body,
        grid_spec=grid_spec,
        out_shape=jax.ShapeDtypeStruct((N_PAD, 8), jnp.int32),
        compiler_params=pltpu.CompilerParams(
            dimension_semantics=("arbitrary", "arbitrary")),
    )(rlo, rhi, clo, chi, x1_pad,
      batch_pad.reshape(N_PAD, 1), batch_pad.reshape(1, N_PAD))
    return idx8[:N_NODES, :K]


def _block(x, W, b, g, be):
    h = jnp.maximum(x @ W.T + b, 0.0)
    return h * (g / jnp.sqrt(1.0 + EPS)) + be


def kernel(pos, batch, edge_index, params):
    p = params
    n = pos.shape[0]
    scales = {k: p['g' + k[1]] / jnp.sqrt(1.0 + EPS)
              for k in ('s1', 's2', 's3', 's4', 's5', 's6', 's7')}

    loops = jnp.arange(n, dtype=edge_index.dtype)
    src = jnp.concatenate([edge_index[0], loops])
    dst = jnp.concatenate([edge_index[1], loops])
    x_i = pos[dst]
    x_j = pos[src]
    m = jnp.concatenate([x_i, x_j - x_i], axis=1)
    m = _block(m, p['W1'], p['b1'], p['g1'], p['be1'])
    m = _block(m, p['W2'], p['b2'], p['g2'], p['be2'])
    m = _block(m, p['W3'], p['b3'], p['g3'], p['be3'])
    x1 = jax.ops.segment_max(m, dst, num_segments=n)

    x1_pad = jnp.pad(x1, ((0, N_PAD - n), (0, 0)))
    batch_pad_1d = jnp.pad(batch.astype(jnp.int32), (0, N_PAD - n),
                           constant_values=N_GRAPHS)
    idx = _knn_topk(x1_pad, batch_pad_1d)
    xj = x1[idx]
    xi = x1[:, None, :]
    m2 = jnp.concatenate([jnp.broadcast_to(xi, xj.shape), xj - xi], axis=2)
    m2 = _block(m2, p['W4'], p['b4'], p['g4'], p['be4'])
    x2 = jnp.max(m2, axis=1)

    h = jnp.concatenate([x1, x2], axis=1)
    h = jnp.pad(h, ((0, N_PAD - n), (0, 0)))
    batch_pad = jnp.pad(batch.astype(jnp.int32), (0, N_PAD - n),
                        constant_values=N_GRAPHS).reshape(N_PAD, 1)
    bt = batch_pad.reshape(N_ROW_TILES, ROW_TILE)
    lo = jnp.min(bt, axis=1).astype(jnp.int32)
    hi = jnp.max(bt, axis=1).astype(jnp.int32)
    z5max = _stage3_pool(h, batch_pad, lo, hi, p['W5'], p['b5'])
    return _head(z5max, p, scales)


# speedup vs baseline: 3.1963x; 1.2184x over previous
import functools

import jax
import jax.numpy as jnp
import numpy as np
from jax import lax
from jax.experimental import pallas as pl
from jax.experimental.pallas import tpu as pltpu
import jax.experimental.pallas.tpu_sc as plsc

N_NODES = 10000
N_GRAPHS = 8
K = 5
EPS = 1e-5

N_PAD = 10240
ROW_TILE = 256
N_ROW_TILES = N_PAD // ROW_TILE


def _stage3_body(lo_ref, hi_ref, x_ref, batch_ref, w5_ref, b5_ref, out_ref):
    i = pl.program_id(0)

    @pl.when(i == 0)
    def _init():
        out_ref[...] = jnp.full_like(out_ref, -jnp.inf)

    z = jax.lax.dot_general(x_ref[...], w5_ref[...], (((1,), (1,)), ((), ())),
                            preferred_element_type=jnp.float32)
    z = z + b5_ref[...]
    lo = lo_ref[i]
    hi = hi_ref[i]
    batch = batch_ref[...]

    def body(g, _):
        mask = batch == g
        zm = jnp.max(jnp.where(mask, z, -jnp.inf), axis=0, keepdims=True)
        cur = out_ref[pl.ds(g, 1), :]
        out_ref[pl.ds(g, 1), :] = jnp.maximum(cur, zm)
        return 0

    jax.lax.fori_loop(lo, hi + 1, body, 0)


def _stage3_pool(x, batch_pad, lo, hi, W5, b5):
    grid_spec = pltpu.PrefetchScalarGridSpec(
        num_scalar_prefetch=2,
        grid=(N_ROW_TILES,),
        in_specs=[
            pl.BlockSpec((ROW_TILE, 192), lambda i, lo, hi: (i, 0)),
            pl.BlockSpec((ROW_TILE, 1), lambda i, lo, hi: (i, 0)),
            pl.BlockSpec((1024, 192), lambda i, lo, hi: (0, 0)),
            pl.BlockSpec((1, 1024), lambda i, lo, hi: (0, 0)),
        ],
        out_specs=pl.BlockSpec((16, 1024), lambda i, lo, hi: (0, 0)),
    )
    out = pl.pallas_call(
        _stage3_body,
        grid_spec=grid_spec,
        out_shape=jax.ShapeDtypeStruct((16, 1024), jnp.float32),
    )(lo, hi, x, batch_pad, W5, b5.reshape(1, 1024))
    return out[:N_GRAPHS]


def _head_body(z_ref, s5_ref, be5_ref, w6_ref, b6_ref, s6_ref, be6_ref,
               w7_ref, b7_ref, s7_ref, be7_ref, w8_ref, b8_ref, out_ref):
    x = jnp.maximum(z_ref[...], 0.0) * s5_ref[...] + be5_ref[...]
    z = jax.lax.dot_general(x, w6_ref[...], (((1,), (1,)), ((), ())),
                            preferred_element_type=jnp.float32) + b6_ref[...]
    x = jnp.maximum(z, 0.0) * s6_ref[...] + be6_ref[...]
    z = jax.lax.dot_general(x, w7_ref[...], (((1,), (1,)), ((), ())),
                            preferred_element_type=jnp.float32) + b7_ref[...]
    x = jnp.maximum(z, 0.0) * s7_ref[...] + be7_ref[...]
    out_ref[...] = jax.lax.dot_general(x, w8_ref[...], (((1,), (1,)), ((), ())),
                                       preferred_element_type=jnp.float32) \
        + b8_ref[...]


def _head(z5max, p, s):
    args = (z5max, s['s5'].reshape(1, -1), p['be5'].reshape(1, -1),
            p['W6'], p['b6'].reshape(1, -1), s['s6'].reshape(1, -1),
            p['be6'].reshape(1, -1),
            p['W7'], p['b7'].reshape(1, -1), s['s7'].reshape(1, -1),
            p['be7'].reshape(1, -1),
            p['W8'], p['b8'].reshape(1, -1))
    return pl.pallas_call(
        _head_body,
        out_shape=jax.ShapeDtypeStruct((N_GRAPHS, 40), jnp.float32),
    )(*args)


E_REAL = N_NODES + 160000
E_PAD = 172032
NW = 32
PER_W_E = E_PAD // NW
GC = 1344
NPW = N_PAD // NW
SCAN_CHUNK = 4096
OC = 8192
RMW_B = 512

_SC_MESH = dict(core_axis_name="c", subcore_axis_name="s",
                num_cores=2, num_subcores=16)


def _ab_body(pos_ref, wa_ref, wb_ref, b1_ref, a_ref, b_ref):
    a_ref[...] = jax.lax.dot_general(
        pos_ref[...], wa_ref[...], (((1,), (0,)), ((), ())),
        preferred_element_type=jnp.float32) + b1_ref[...]
    b_ref[...] = jax.lax.dot_general(
        pos_ref[...], wb_ref[...], (((1,), (0,)), ((), ())),
        preferred_element_type=jnp.float32)


def _ab_kernel(pos_pad, wa8, wb8, b1):
    return pl.pallas_call(
        _ab_body,
        grid=(N_PAD // 2048,),
        in_specs=[pl.BlockSpec((2048, 8), lambda i: (i, 0)),
                  pl.BlockSpec((8, 64), lambda i: (0, 0)),
                  pl.BlockSpec((8, 64), lambda i: (0, 0)),
                  pl.BlockSpec((1, 64), lambda i: (0, 0))],
        out_specs=[pl.BlockSpec((2048, 64), lambda i: (i, 0)),
                   pl.BlockSpec((2048, 64), lambda i: (i, 0))],
        out_shape=[jax.ShapeDtypeStruct((N_PAD, 64), jnp.float32),
                   jax.ShapeDtypeStruct((N_PAD, 64), jnp.float32)],
    )(pos_pad, wa8, wb8, b1.reshape(1, 64))


def _edge_gather_body(a_hbm, b_hbm, src_hbm, dst_hbm, e1_hbm,
                      sidx, didx, rows, sem):
    wid = lax.axis_index("s") * 2 + lax.axis_index("c")
    base = wid * PER_W_E

    def chunk(ci, _):
        off = base + ci * GC
        pltpu.sync_copy(dst_hbm.at[pl.ds(off, GC)], didx)
        pltpu.sync_copy(src_hbm.at[pl.ds(off, GC)], sidx)
        pltpu.async_copy(a_hbm.at[didx], rows, sem).wait()
        pltpu.async_copy(b_hbm.at[sidx], rows, sem, add=True).wait()
        pltpu.sync_copy(rows, e1_hbm.at[pl.ds(off, GC)])
        return 0

    lax.fori_loop(0, PER_W_E // GC, chunk, 0)


def _edge_gather(A, B, src_pad, dst_pad):
    fn = pl.kernel(
        _edge_gather_body,
        out_type=jax.ShapeDtypeStruct((E_PAD, 64), jnp.float32),
        mesh=plsc.VectorSubcoreMesh(**_SC_MESH),
        scratch_types=[pltpu.VMEM((GC,), jnp.int32),
                       pltpu.VMEM((GC,), jnp.int32),
                       pltpu.VMEM((GC, 64), jnp.float32),
                       pltpu.SemaphoreType.DMA],
        compiler_params=pltpu.CompilerParams(use_tc_tiling_on_sc=False),
    )
    return fn(A, B, src_pad, dst_pad)


def _edge_mlp_body(e1_ref, w2_ref, b2_ref, w3_ref, b3_ref, sc_ref, m_ref):
    i = pl.program_id(0)
    s1 = sc_ref[0:1, :]
    be1 = sc_ref[1:2, :]
    s2 = sc_ref[2:3, :]
    be2 = sc_ref[3:4, :]
    s3 = sc_ref[4:5, :]
    be3 = sc_ref[5:6, :]
    h = jnp.maximum(e1_ref[...], 0.0) * s1 + be1
    z = jax.lax.dot_general(h, w2_ref[...], (((1,), (1,)), ((), ())),
                            preferred_element_type=jnp.float32) + b2_ref[...]
    h = jnp.maximum(z, 0.0) * s2 + be2
    z = jax.lax.dot_general(h, w3_ref[...], (((1,), (1,)), ((), ())),
                            preferred_element_type=jnp.float32) + b3_ref[...]
    h = jnp.maximum(z, 0.0) * s3 + be3
    gid = i * 512 + jax.lax.broadcasted_iota(jnp.int32, (512, 1), 0)
    m_ref[...] = jnp.where(gid < E_REAL, h, -jnp.inf)


def _edge_mlp(E1, p, s):
    sc = jnp.stack([s['s1'], p['be1'], s['s2'], p['be2'], s['s3'], p['be3']])
    return pl.pallas_call(
        _edge_mlp_body,
        grid=(E_PAD // 512,),
        in_specs=[pl.BlockSpec((512, 64), lambda i: (i, 0)),
                  pl.BlockSpec((64, 64), lambda i: (0, 0)),
                  pl.BlockSpec((1, 64), lambda i: (0, 0)),
                  pl.BlockSpec((64, 64), lambda i: (0, 0)),
                  pl.BlockSpec((1, 64), lambda i: (0, 0)),
                  pl.BlockSpec((6, 64), lambda i: (0, 0))],
        out_specs=pl.BlockSpec((512, 64), lambda i: (i, 0)),
        out_shape=jax.ShapeDtypeStruct((E_PAD, 64), jnp.float32),
    )(E1, p['W2'], p['b2'].reshape(1, 64), p['W3'], p['b3'].reshape(1, 64), sc)


def _scatter_max_body(m_hbm, dst_hbm, x1_hbm,
                      x1b, dstv, oids, odst, grows, dsts, sem):
    wid = lax.axis_index("s") * 2 + lax.axis_index("c")
    lo = wid * NPW
    hi = lo + NPW

    def initr(r, _):
        for q in range(4):
            x1b[r, pl.ds(q * 16, 16)] = jnp.full((16,), -jnp.inf, jnp.float32)
        return 0
    lax.fori_loop(0, NPW, initr, 0)

    def inito(r, _):
        def inner(k, _):
            oids[r, pl.ds(k * 16, 16)] = jnp.full((16,), E_PAD - 1, jnp.int32)
            odst[r, pl.ds(k * 16, 16)] = jnp.full((16,), 0, jnp.int32) + lo
            return 0
        lax.fori_loop(0, RMW_B // 16, inner, 0)
        return 0
    lax.fori_loop(0, OC // RMW_B + 1, inito, 0)

    def chunkA(c, offv):
        pltpu.sync_copy(dst_hbm.at[pl.ds(c * SCAN_CHUNK, SCAN_CHUNK)], dstv)

        def vec(k, offv):
            v = dstv[pl.ds(k * 16, 16)]
            m = jnp.logical_and(v >= lo, v < hi)
            cs = plsc.cumsum(m.astype(jnp.int32))
            eid = lax.iota(jnp.int32, 16) + (c * SCAN_CHUNK + k * 16)
            pos = jnp.minimum(jnp.where(m, offv + cs - 1, OC), OC)
            prow = jax.lax.shift_right_logical(pos, 9)
            pcol = jnp.bitwise_and(pos, RMW_B - 1)
            plsc.store_scatter(oids, [prow, pcol], eid)
            plsc.store_scatter(odst, [prow, pcol], v)
            return offv + plsc.all_reduce_population_count(m)

        return lax.fori_loop(0, SCAN_CHUNK // 16, vec, offv)

    lax.fori_loop(0, E_PAD // SCAN_CHUNK, chunkA, jnp.zeros((16,), jnp.int32))

    def batch(bb, _):
        pltpu.async_copy(m_hbm.at[oids.at[bb]], grows, sem).wait()
        pltpu.sync_copy(odst.at[bb], dsts)

        def rmw(e, _):
            lr = dsts[e] - lo
            for q in range(4):
                a = x1b[lr, pl.ds(q * 16, 16)]
                b = grows[e, pl.ds(q * 16, 16)]
                x1b[lr, pl.ds(q * 16, 16)] = jnp.maximum(a, b)
            return 0

        lax.fori_loop(0, RMW_B, rmw, 0)
        return 0

    lax.fori_loop(0, OC // RMW_B, batch, 0)

    def zeror(r, _):
        for q in range(4):
            x1b[r, pl.ds(q * 16, 16)] = jnp.zeros((16,), jnp.float32)
        return 0
    lax.fori_loop(jnp.maximum(lo, N_NODES) - lo, NPW, zeror, 0)

    pltpu.sync_copy(x1b, x1_hbm.at[pl.ds(lo, NPW)])


def _scatter_max(M, dst_pad):
    fn = pl.kernel(
        _scatter_max_body,
        out_type=jax.ShapeDtypeStruct((N_PAD, 64), jnp.float32),
        mesh=plsc.VectorSubcoreMesh(**_SC_MESH),
        scratch_types=[pltpu.VMEM((NPW, 64), jnp.float32),
                       pltpu.VMEM((SCAN_CHUNK,), jnp.int32),
                       pltpu.VMEM((OC // RMW_B + 1, RMW_B), jnp.int32),
                       pltpu.VMEM((OC // RMW_B + 1, RMW_B), jnp.int32),
                       pltpu.VMEM((RMW_B, 64), jnp.float32),
                       pltpu.SMEM((RMW_B,), jnp.int32),
                       pltpu.SemaphoreType.DMA],
        compiler_params=pltpu.CompilerParams(use_tc_tiling_on_sc=False),
    )
    return fn(M, dst_pad)


KNN_RT = 128
KNN_CT = 512
KNN_NI = N_PAD // KNN_RT
KNN_NJ = N_PAD // KNN_CT
IMAX = np.int32(0x7FFFFFFF)


def _knn_body(rlo, rhi, clo, chi, xall_ref, batchr_ref, batchc_ref, out_ref,
              td, ti):
    i = pl.program_id(0)
    j = pl.program_id(1)

    @pl.when(j == 0)
    def _init():
        td[...] = jnp.full_like(td, jnp.inf)
        ti[...] = jnp.full_like(ti, IMAX)

    active = jnp.logical_and(rhi[i] >= clo[j], rlo[i] <= chi[j])

    @pl.when(active)
    def _merge():
        xr = xall_ref[pl.ds(i * KNN_RT, KNN_RT), :]
        xc = xall_ref[pl.ds(j * KNN_CT, KNN_CT), :]
        r = jax.lax.dot_general(xr, xc, (((1,), (1,)), ((), ())),
                                preferred_element_type=jnp.float32)
        sqc = jax.lax.dot_general(jnp.ones((1, 64), jnp.float32), xc * xc,
                                  (((1,), (1,)), ((), ())),
                                  preferred_element_type=jnp.float32)
        r = sqc - 2.0 * r
        bm = batchr_ref[...] != batchc_ref[...]
        r = jnp.where(bm, jnp.inf, r)
        colid = jax.lax.broadcasted_iota(jnp.int32, (KNN_RT, KNN_CT), 1) \
            + j * KNN_CT
        cd = jnp.concatenate([td[...], r], axis=1)
        ci = jnp.concatenate([ti[...], colid], axis=1)
        for k in range(K):
            m = jnp.min(cd, axis=1, keepdims=True)
            sel = jnp.min(jnp.where(cd == m, ci, IMAX), axis=1, keepdims=True)
            td[:, k:k + 1] = m
            ti[:, k:k + 1] = sel
            cd = jnp.where(ci == sel, jnp.inf, cd)

    @pl.when(j == KNN_NJ - 1)
    def _emit():
        out_ref[...] = ti[:, :8]


def _knn_topk(x1_pad, batch_pad):
    bt_r = batch_pad.reshape(KNN_NI, KNN_RT)
    bt_c = batch_pad.reshape(KNN_NJ, KNN_CT)
    rlo = bt_r[:, 0].astype(jnp.int32)
    rhi = bt_r[:, -1].astype(jnp.int32)
    clo = bt_c[:, 0].astype(jnp.int32)
    chi = bt_c[:, -1].astype(jnp.int32)
    grid_spec = pltpu.PrefetchScalarGridSpec(
        num_scalar_prefetch=4,
        grid=(KNN_NI, KNN_NJ),
        in_specs=[
            pl.BlockSpec((N_PAD, 64), lambda i, j, *_: (0, 0)),
            pl.BlockSpec((KNN_RT, 1), lambda i, j, *_: (i, 0)),
            pl.BlockSpec((1, KNN_CT), lambda i, j, *_: (0, j)),
        ],
        out_specs=pl.BlockSpec((KNN_RT, 8), lambda i, j, *_: (i, 0)),
        scratch_shapes=[pltpu.VMEM((KNN_RT, 128), jnp.float32),
                        pltpu.VMEM((KNN_RT, 128), jnp.int32)],
    )
    idx8 = pl.pallas_call(
        _knn_body,
        grid_spec=grid_spec,
        out_shape=jax.ShapeDtypeStruct((N_PAD, 8), jnp.int32),
        compiler_params=pltpu.CompilerParams(
            dimension_semantics=("arbitrary", "arbitrary")),
    )(rlo, rhi, clo, chi, x1_pad,
      batch_pad.reshape(N_PAD, 1), batch_pad.reshape(1, N_PAD))
    return idx8[:N_NODES, :K]


def _block(x, W, b, g, be):
    h = jnp.maximum(x @ W.T + b, 0.0)
    return h * (g / jnp.sqrt(1.0 + EPS)) + be


def kernel(pos, batch, edge_index, params):
    p = params
    n = pos.shape[0]
    scales = {k: p['g' + k[1]] / jnp.sqrt(1.0 + EPS)
              for k in ('s1', 's2', 's3', 's4', 's5', 's6', 's7')}

    loops = jnp.arange(n, dtype=edge_index.dtype)
    src = jnp.concatenate([edge_index[0], loops]).astype(jnp.int32)
    dst = jnp.concatenate([edge_index[1], loops]).astype(jnp.int32)
    src_pad = jnp.pad(src, (0, E_PAD - E_REAL))
    dst_pad = jnp.pad(dst, (0, E_PAD - E_REAL), constant_values=N_PAD - 1)
    pos_pad = jnp.pad(pos, ((0, N_PAD - n), (0, 5)))
    W1 = p['W1']
    wa8 = jnp.pad((W1[:, :3] - W1[:, 3:]).T, ((0, 5), (0, 0)))
    wb8 = jnp.pad(W1[:, 3:].T, ((0, 5), (0, 0)))
    A, B = _ab_kernel(pos_pad, wa8, wb8, p['b1'])
    E1 = _edge_gather(A, B, src_pad, dst_pad)
    M = _edge_mlp(E1, p, scales)
    x1 = jax.ops.segment_max(M[:E_REAL], dst, num_segments=n)
    x1_pad_sc = jnp.pad(x1, ((0, N_PAD - n), (0, 0)))

    x1_pad = x1_pad_sc
    batch_pad_1d = jnp.pad(batch.astype(jnp.int32), (0, N_PAD - n),
                           constant_values=N_GRAPHS)
    idx = _knn_topk(x1_pad, batch_pad_1d)
    xj = x1[idx]
    xi = x1[:, None, :]
    m2 = jnp.concatenate([jnp.broadcast_to(xi, xj.shape), xj - xi], axis=2)
    m2 = _block(m2, p['W4'], p['b4'], p['g4'], p['be4'])
    x2 = jnp.max(m2, axis=1)

    h = jnp.concatenate([x1, x2], axis=1)
    h = jnp.pad(h, ((0, N_PAD - n), (0, 0)))
    batch_pad = jnp.pad(batch.astype(jnp.int32), (0, N_PAD - n),
                        constant_values=N_GRAPHS).reshape(N_PAD, 1)
    bt = batch_pad.reshape(N_ROW_TILES, ROW_TILE)
    lo = jnp.min(bt, axis=1).astype(jnp.int32)
    hi = jnp.max(bt, axis=1).astype(jnp.int32)
    z5max = _stage3_pool(h, batch_pad, lo, hi, p['W5'], p['b5'])
    return _head(z5max, p, scales)


# speedup vs baseline: 3.3389x; 1.0446x over previous
import functools

import jax
import jax.numpy as jnp
import numpy as np
from jax import lax
from jax.experimental import pallas as pl
from jax.experimental.pallas import tpu as pltpu
import jax.experimental.pallas.tpu_sc as plsc

N_NODES = 10000
N_GRAPHS = 8
K = 5
EPS = 1e-5

N_PAD = 10240
ROW_TILE = 256
N_ROW_TILES = N_PAD // ROW_TILE


def _stage3_body(lo_ref, hi_ref, x1_ref, z4_ref, batch_ref, w5a_ref, w5b_ref,
                 b5_ref, s4_ref, be4_ref, out_ref):
    i = pl.program_id(0)

    @pl.when(i == 0)
    def _init():
        out_ref[...] = jnp.full_like(out_ref, -jnp.inf)

    x2 = jnp.maximum(z4_ref[...], 0.0) * s4_ref[...] + be4_ref[...]
    z = jax.lax.dot_general(x1_ref[...], w5a_ref[...], (((1,), (1,)), ((), ())),
                            preferred_element_type=jnp.float32)
    z = z + jax.lax.dot_general(x2, w5b_ref[...], (((1,), (1,)), ((), ())),
                                preferred_element_type=jnp.float32)
    z = z + b5_ref[...]
    lo = lo_ref[i]
    hi = hi_ref[i]
    batch = batch_ref[...]

    def body(g, _):
        mask = batch == g
        zm = jnp.max(jnp.where(mask, z, -jnp.inf), axis=0, keepdims=True)
        cur = out_ref[pl.ds(g, 1), :]
        out_ref[pl.ds(g, 1), :] = jnp.maximum(cur, zm)
        return 0

    jax.lax.fori_loop(lo, hi + 1, body, 0)


def _stage3_pool(x1_pad, z4, batch_pad, lo, hi, W5, b5, s4, be4):
    grid_spec = pltpu.PrefetchScalarGridSpec(
        num_scalar_prefetch=2,
        grid=(N_ROW_TILES,),
        in_specs=[
            pl.BlockSpec((ROW_TILE, 64), lambda i, lo, hi: (i, 0)),
            pl.BlockSpec((ROW_TILE, 128), lambda i, lo, hi: (i, 0)),
            pl.BlockSpec((ROW_TILE, 1), lambda i, lo, hi: (i, 0)),
            pl.BlockSpec((1024, 64), lambda i, lo, hi: (0, 0)),
            pl.BlockSpec((1024, 128), lambda i, lo, hi: (0, 0)),
            pl.BlockSpec((1, 1024), lambda i, lo, hi: (0, 0)),
            pl.BlockSpec((1, 128), lambda i, lo, hi: (0, 0)),
            pl.BlockSpec((1, 128), lambda i, lo, hi: (0, 0)),
        ],
        out_specs=pl.BlockSpec((16, 1024), lambda i, lo, hi: (0, 0)),
    )
    out = pl.pallas_call(
        _stage3_body,
        grid_spec=grid_spec,
        out_shape=jax.ShapeDtypeStruct((16, 1024), jnp.float32),
    )(lo, hi, x1_pad, z4, batch_pad, W5[:, :64], W5[:, 64:],
      b5.reshape(1, 1024), s4.reshape(1, 128), be4.reshape(1, 128))
    return out[:N_GRAPHS]


def _head_body(z_ref, s5_ref, be5_ref, w6_ref, b6_ref, s6_ref, be6_ref,
               w7_ref, b7_ref, s7_ref, be7_ref, w8_ref, b8_ref, out_ref):
    x = jnp.maximum(z_ref[...], 0.0) * s5_ref[...] + be5_ref[...]
    z = jax.lax.dot_general(x, w6_ref[...], (((1,), (1,)), ((), ())),
                            preferred_element_type=jnp.float32) + b6_ref[...]
    x = jnp.maximum(z, 0.0) * s6_ref[...] + be6_ref[...]
    z = jax.lax.dot_general(x, w7_ref[...], (((1,), (1,)), ((), ())),
                            preferred_element_type=jnp.float32) + b7_ref[...]
    x = jnp.maximum(z, 0.0) * s7_ref[...] + be7_ref[...]
    out_ref[...] = jax.lax.dot_general(x, w8_ref[...], (((1,), (1,)), ((), ())),
                                       preferred_element_type=jnp.float32) \
        + b8_ref[...]


def _head(z5max, p, s):
    args = (z5max, s['s5'].reshape(1, -1), p['be5'].reshape(1, -1),
            p['W6'], p['b6'].reshape(1, -1), s['s6'].reshape(1, -1),
            p['be6'].reshape(1, -1),
            p['W7'], p['b7'].reshape(1, -1), s['s7'].reshape(1, -1),
            p['be7'].reshape(1, -1),
            p['W8'], p['b8'].reshape(1, -1))
    return pl.pallas_call(
        _head_body,
        out_shape=jax.ShapeDtypeStruct((N_GRAPHS, 40), jnp.float32),
    )(*args)


E_REAL = N_NODES + 160000
E_PAD = 172032
NW = 32
PER_W_E = E_PAD // NW
GC = 1344
NPW = N_PAD // NW
SCAN_CHUNK = 4096
OC = 8192
RMW_B = 512

_SC_MESH = dict(core_axis_name="c", subcore_axis_name="s",
                num_cores=2, num_subcores=16)


def _ab_body(pos_ref, wa_ref, wb_ref, b1_ref, a_ref, b_ref):
    a_ref[...] = jax.lax.dot_general(
        pos_ref[...], wa_ref[...], (((1,), (0,)), ((), ())),
        preferred_element_type=jnp.float32) + b1_ref[...]
    b_ref[...] = jax.lax.dot_general(
        pos_ref[...], wb_ref[...], (((1,), (0,)), ((), ())),
        preferred_element_type=jnp.float32)


def _ab_kernel(pos_pad, wa8, wb8, b1):
    return pl.pallas_call(
        _ab_body,
        grid=(N_PAD // 2048,),
        in_specs=[pl.BlockSpec((2048, 8), lambda i: (i, 0)),
                  pl.BlockSpec((8, 64), lambda i: (0, 0)),
                  pl.BlockSpec((8, 64), lambda i: (0, 0)),
                  pl.BlockSpec((1, 64), lambda i: (0, 0))],
        out_specs=[pl.BlockSpec((2048, 64), lambda i: (i, 0)),
                   pl.BlockSpec((2048, 64), lambda i: (i, 0))],
        out_shape=[jax.ShapeDtypeStruct((N_PAD, 64), jnp.float32),
                   jax.ShapeDtypeStruct((N_PAD, 64), jnp.float32)],
    )(pos_pad, wa8, wb8, b1.reshape(1, 64))


def _edge_gather_body(a_hbm, b_hbm, src_hbm, dst_hbm, e1_hbm,
                      sidx, didx, rows, sem):
    wid = lax.axis_index("s") * 2 + lax.axis_index("c")
    base = wid * PER_W_E

    def chunk(ci, _):
        off = base + ci * GC
        pltpu.sync_copy(dst_hbm.at[pl.ds(off, GC)], didx)
        pltpu.sync_copy(src_hbm.at[pl.ds(off, GC)], sidx)
        pltpu.async_copy(a_hbm.at[didx], rows, sem).wait()
        pltpu.async_copy(b_hbm.at[sidx], rows, sem, add=True).wait()
        pltpu.sync_copy(rows, e1_hbm.at[pl.ds(off, GC)])
        return 0

    lax.fori_loop(0, PER_W_E // GC, chunk, 0)


def _edge_gather(A, B, src_pad, dst_pad):
    fn = pl.kernel(
        _edge_gather_body,
        out_type=jax.ShapeDtypeStruct((E_PAD, 64), jnp.float32),
        mesh=plsc.VectorSubcoreMesh(**_SC_MESH),
        scratch_types=[pltpu.VMEM((GC,), jnp.int32),
                       pltpu.VMEM((GC,), jnp.int32),
                       pltpu.VMEM((GC, 64), jnp.float32),
                       pltpu.SemaphoreType.DMA],
        compiler_params=pltpu.CompilerParams(use_tc_tiling_on_sc=False),
    )
    return fn(A, B, src_pad, dst_pad)


def _edge_mlp_body(e1_ref, w2_ref, b2_ref, w3_ref, b3_ref, sc_ref, m_ref):
    i = pl.program_id(0)
    s1 = sc_ref[0:1, :]
    be1 = sc_ref[1:2, :]
    s2 = sc_ref[2:3, :]
    be2 = sc_ref[3:4, :]
    s3 = sc_ref[4:5, :]
    be3 = sc_ref[5:6, :]
    h = jnp.maximum(e1_ref[...], 0.0) * s1 + be1
    z = jax.lax.dot_general(h, w2_ref[...], (((1,), (1,)), ((), ())),
                            preferred_element_type=jnp.float32) + b2_ref[...]
    h = jnp.maximum(z, 0.0) * s2 + be2
    z = jax.lax.dot_general(h, w3_ref[...], (((1,), (1,)), ((), ())),
                            preferred_element_type=jnp.float32) + b3_ref[...]
    h = jnp.maximum(z, 0.0) * s3 + be3
    gid = i * 512 + jax.lax.broadcasted_iota(jnp.int32, (512, 1), 0)
    m_ref[...] = jnp.where(gid < E_REAL, h, -jnp.inf)


def _edge_mlp(E1, p, s):
    sc = jnp.stack([s['s1'], p['be1'], s['s2'], p['be2'], s['s3'], p['be3']])
    return pl.pallas_call(
        _edge_mlp_body,
        grid=(E_PAD // 512,),
        in_specs=[pl.BlockSpec((512, 64), lambda i: (i, 0)),
                  pl.BlockSpec((64, 64), lambda i: (0, 0)),
                  pl.BlockSpec((1, 64), lambda i: (0, 0)),
                  pl.BlockSpec((64, 64), lambda i: (0, 0)),
                  pl.BlockSpec((1, 64), lambda i: (0, 0)),
                  pl.BlockSpec((6, 64), lambda i: (0, 0))],
        out_specs=pl.BlockSpec((512, 64), lambda i: (i, 0)),
        out_shape=jax.ShapeDtypeStruct((E_PAD, 64), jnp.float32),
    )(E1, p['W2'], p['b2'].reshape(1, 64), p['W3'], p['b3'].reshape(1, 64), sc)


def _scatter_max_body(m_hbm, dst_hbm, x1_hbm,
                      x1b, dstv, oids, odst, grows, dsts, sem):
    wid = lax.axis_index("s") * 2 + lax.axis_index("c")
    lo = wid * NPW
    hi = lo + NPW

    def initr(r, _):
        for q in range(4):
            x1b[r, pl.ds(q * 16, 16)] = jnp.full((16,), -jnp.inf, jnp.float32)
        return 0
    lax.fori_loop(0, NPW, initr, 0)

    def inito(r, _):
        def inner(k, _):
            oids[r, pl.ds(k * 16, 16)] = jnp.full((16,), E_PAD - 1, jnp.int32)
            odst[r, pl.ds(k * 16, 16)] = jnp.full((16,), 0, jnp.int32) + lo
            return 0
        lax.fori_loop(0, RMW_B // 16, inner, 0)
        return 0
    lax.fori_loop(0, OC // RMW_B + 1, inito, 0)

    def chunkA(c, offv):
        pltpu.sync_copy(dst_hbm.at[pl.ds(c * SCAN_CHUNK, SCAN_CHUNK)], dstv)

        def vec(k, offv):
            v = dstv[pl.ds(k * 16, 16)]
            m = jnp.logical_and(v >= lo, v < hi)
            cs = plsc.cumsum(m.astype(jnp.int32))
            eid = lax.iota(jnp.int32, 16) + (c * SCAN_CHUNK + k * 16)
            pos = jnp.minimum(jnp.where(m, offv + cs - 1, OC), OC)
            prow = jax.lax.shift_right_logical(pos, 9)
            pcol = jnp.bitwise_and(pos, RMW_B - 1)
            plsc.store_scatter(oids, [prow, pcol], eid)
            plsc.store_scatter(odst, [prow, pcol], v)
            return offv + plsc.all_reduce_population_count(m)

        return lax.fori_loop(0, SCAN_CHUNK // 16, vec, offv)

    lax.fori_loop(0, E_PAD // SCAN_CHUNK, chunkA, jnp.zeros((16,), jnp.int32))

    def batch(bb, _):
        pltpu.async_copy(m_hbm.at[oids.at[bb]], grows, sem).wait()
        pltpu.sync_copy(odst.at[bb], dsts)

        def rmw(e, _):
            lr = dsts[e] - lo
            for q in range(4):
                a = x1b[lr, pl.ds(q * 16, 16)]
                b = grows[e, pl.ds(q * 16, 16)]
                x1b[lr, pl.ds(q * 16, 16)] = jnp.maximum(a, b)
            return 0

        lax.fori_loop(0, RMW_B, rmw, 0)
        return 0

    lax.fori_loop(0, OC // RMW_B, batch, 0)

    def zeror(r, _):
        for q in range(4):
            x1b[r, pl.ds(q * 16, 16)] = jnp.zeros((16,), jnp.float32)
        return 0
    lax.fori_loop(jnp.maximum(lo, N_NODES) - lo, NPW, zeror, 0)

    pltpu.sync_copy(x1b, x1_hbm.at[pl.ds(lo, NPW)])


def _scatter_max(M, dst_pad):
    fn = pl.kernel(
        _scatter_max_body,
        out_type=jax.ShapeDtypeStruct((N_PAD, 64), jnp.float32),
        mesh=plsc.VectorSubcoreMesh(**_SC_MESH),
        scratch_types=[pltpu.VMEM((NPW, 64), jnp.float32),
                       pltpu.VMEM((SCAN_CHUNK,), jnp.int32),
                       pltpu.VMEM((OC // RMW_B + 1, RMW_B), jnp.int32),
                       pltpu.VMEM((OC // RMW_B + 1, RMW_B), jnp.int32),
                       pltpu.VMEM((RMW_B, 64), jnp.float32),
                       pltpu.SMEM((RMW_B,), jnp.int32),
                       pltpu.SemaphoreType.DMA],
        compiler_params=pltpu.CompilerParams(use_tc_tiling_on_sc=False),
    )
    return fn(M, dst_pad)


KNN_RT = 128
KNN_CT = 512
KNN_NI = N_PAD // KNN_RT
KNN_NJ = N_PAD // KNN_CT
IMAX = np.int32(0x7FFFFFFF)


def _knn_body(rlo, rhi, clo, chi, xall_ref, batchr_ref, batchc_ref,
              w4d_ref, w4b_ref, b4_ref, out_ref, p_ref, q_ref, td, ti):
    i = pl.program_id(0)
    j = pl.program_id(1)

    @pl.when(j == 0)
    def _init():
        td[...] = jnp.full_like(td, jnp.inf)
        ti[...] = jnp.full_like(ti, IMAX)
        xr0 = xall_ref[pl.ds(i * KNN_RT, KNN_RT), :]
        p_ref[...] = jax.lax.dot_general(
            xr0, w4d_ref[...], (((1,), (1,)), ((), ())),
            preferred_element_type=jnp.float32) + b4_ref[...]
        q_ref[...] = jax.lax.dot_general(
            xr0, w4b_ref[...], (((1,), (1,)), ((), ())),
            preferred_element_type=jnp.float32)

    active = jnp.logical_and(rhi[i] >= clo[j], rlo[i] <= chi[j])

    @pl.when(active)
    def _merge():
        xr = xall_ref[pl.ds(i * KNN_RT, KNN_RT), :]
        xc = xall_ref[pl.ds(j * KNN_CT, KNN_CT), :]
        r = jax.lax.dot_general(xr, xc, (((1,), (1,)), ((), ())),
                                preferred_element_type=jnp.float32)
        sqc = jax.lax.dot_general(jnp.ones((1, 64), jnp.float32), xc * xc,
                                  (((1,), (1,)), ((), ())),
                                  preferred_element_type=jnp.float32)
        r = sqc - 2.0 * r
        bm = batchr_ref[...] != batchc_ref[...]
        r = jnp.where(bm, jnp.inf, r)
        colid = jax.lax.broadcasted_iota(jnp.int32, (KNN_RT, KNN_CT), 1) \
            + j * KNN_CT
        cd = jnp.concatenate([td[...], r], axis=1)
        ci = jnp.concatenate([ti[...], colid], axis=1)
        for k in range(K):
            m = jnp.min(cd, axis=1, keepdims=True)
            sel = jnp.min(jnp.where(cd == m, ci, IMAX), axis=1, keepdims=True)
            td[:, k:k + 1] = m
            ti[:, k:k + 1] = sel
            cd = jnp.where(ci == sel, jnp.inf, cd)

    @pl.when(j == KNN_NJ - 1)
    def _emit():
        out_ref[...] = ti[:, :8]


def _knn_topk(x1_pad, batch_pad, W4, b4):
    bt_r = batch_pad.reshape(KNN_NI, KNN_RT)
    bt_c = batch_pad.reshape(KNN_NJ, KNN_CT)
    rlo = bt_r[:, 0].astype(jnp.int32)
    rhi = bt_r[:, -1].astype(jnp.int32)
    clo = bt_c[:, 0].astype(jnp.int32)
    chi = bt_c[:, -1].astype(jnp.int32)
    w4d = W4[:, :64] - W4[:, 64:]
    w4b = W4[:, 64:]
    grid_spec = pltpu.PrefetchScalarGridSpec(
        num_scalar_prefetch=4,
        grid=(KNN_NI, KNN_NJ),
        in_specs=[
            pl.BlockSpec((N_PAD, 64), lambda i, j, *_: (0, 0)),
            pl.BlockSpec((KNN_RT, 1), lambda i, j, *_: (i, 0)),
            pl.BlockSpec((1, KNN_CT), lambda i, j, *_: (0, j)),
            pl.BlockSpec((128, 64), lambda i, j, *_: (0, 0)),
            pl.BlockSpec((128, 64), lambda i, j, *_: (0, 0)),
            pl.BlockSpec((1, 128), lambda i, j, *_: (0, 0)),
        ],
        out_specs=[pl.BlockSpec((KNN_RT, 8), lambda i, j, *_: (i, 0)),
                   pl.BlockSpec((KNN_RT, 128), lambda i, j, *_: (i, 0)),
                   pl.BlockSpec((KNN_RT, 128), lambda i, j, *_: (i, 0))],
        scratch_shapes=[pltpu.VMEM((KNN_RT, 128), jnp.float32),
                        pltpu.VMEM((KNN_RT, 128), jnp.int32)],
    )
    idx8, P, Q = pl.pallas_call(
        _knn_body,
        grid_spec=grid_spec,
        out_shape=[jax.ShapeDtypeStruct((N_PAD, 8), jnp.int32),
                   jax.ShapeDtypeStruct((N_PAD, 128), jnp.float32),
                   jax.ShapeDtypeStruct((N_PAD, 128), jnp.float32)],
        compiler_params=pltpu.CompilerParams(
            dimension_semantics=("arbitrary", "arbitrary")),
    )(rlo, rhi, clo, chi, x1_pad,
      batch_pad.reshape(N_PAD, 1), batch_pad.reshape(1, N_PAD),
      w4d, w4b, b4.reshape(1, 128))
    return idx8[:, :K], P, Q


C2_CH = 64
C2_NCH = NPW // C2_CH


def _conv2_body(p_hbm, q_hbm, idx_hbm, z4_hbm, pvm, qvm, ivm, zvm, sem):
    wid = lax.axis_index("s") * 2 + lax.axis_index("c")
    lo = wid * NPW

    def chunk(c, _):
        nb = lo + c * C2_CH
        pltpu.sync_copy(p_hbm.at[pl.ds(nb * 128, C2_CH * 128)], pvm)
        pltpu.sync_copy(idx_hbm.at[pl.ds(nb * K, C2_CH * K)], ivm)
        pltpu.async_copy(q_hbm.at[ivm], qvm, sem).wait()

        def node(nn, _):
            rowp = jnp.full((16,), 0, jnp.int32) + nn
            for q in range(8):
                col = lax.iota(jnp.int32, 16) + q * 16
                qacc = plsc.load_gather(qvm, [rowp * K, col])
                for k in range(1, K):
                    rowq = rowp * K + k
                    qacc = jnp.maximum(qacc, plsc.load_gather(qvm,
                                                              [rowq, col]))
                acc = plsc.load_gather(pvm, [rowp * 128 + col]) + qacc
                plsc.store_scatter(zvm, [rowp * 128 + col], acc)
            return 0

        lax.fori_loop(0, C2_CH, node, 0)
        pltpu.sync_copy(zvm, z4_hbm.at[pl.ds(nb * 128, C2_CH * 128)])
        return 0

    lax.fori_loop(0, C2_NCH, chunk, 0)


def _conv2_sc(P, Q, idx_flat):
    fn = pl.kernel(
        _conv2_body,
        out_type=jax.ShapeDtypeStruct((N_PAD * 128,), jnp.float32),
        mesh=plsc.VectorSubcoreMesh(**_SC_MESH),
        scratch_types=[pltpu.VMEM((C2_CH * 128,), jnp.float32),
                       pltpu.VMEM((C2_CH * K, 128), jnp.float32),
                       pltpu.VMEM((C2_CH * K,), jnp.int32),
                       pltpu.VMEM((C2_CH * 128,), jnp.float32),
                       pltpu.SemaphoreType.DMA],
        compiler_params=pltpu.CompilerParams(use_tc_tiling_on_sc=False,
                                             needs_layout_passes=False),
    )
    return fn(P.reshape(-1), Q, idx_flat).reshape(N_PAD, 128)


def _block(x, W, b, g, be):
    h = jnp.maximum(x @ W.T + b, 0.0)
    return h * (g / jnp.sqrt(1.0 + EPS)) + be


def kernel(pos, batch, edge_index, params):
    p = params
    n = pos.shape[0]
    scales = {k: p['g' + k[1]] / jnp.sqrt(1.0 + EPS)
              for k in ('s1', 's2', 's3', 's4', 's5', 's6', 's7')}

    loops = jnp.arange(n, dtype=edge_index.dtype)
    src = jnp.concatenate([edge_index[0], loops]).astype(jnp.int32)
    dst = jnp.concatenate([edge_index[1], loops]).astype(jnp.int32)
    src_pad = jnp.pad(src, (0, E_PAD - E_REAL))
    dst_pad = jnp.pad(dst, (0, E_PAD - E_REAL), constant_values=N_PAD - 1)
    pos_pad = jnp.pad(pos, ((0, N_PAD - n), (0, 5)))
    W1 = p['W1']
    wa8 = jnp.pad((W1[:, :3] - W1[:, 3:]).T, ((0, 5), (0, 0)))
    wb8 = jnp.pad(W1[:, 3:].T, ((0, 5), (0, 0)))
    A, B = _ab_kernel(pos_pad, wa8, wb8, p['b1'])
    E1 = _edge_gather(A, B, src_pad, dst_pad)
    M = _edge_mlp(E1, p, scales)
    x1 = jax.ops.segment_max(M[:E_REAL], dst, num_segments=n)
    x1_pad_sc = jnp.pad(x1, ((0, N_PAD - n), (0, 0)))

    x1_pad = x1_pad_sc
    batch_pad_1d = jnp.pad(batch.astype(jnp.int32), (0, N_PAD - n),
                           constant_values=N_GRAPHS)
    idx, P, Q = _knn_topk(x1_pad, batch_pad_1d, p['W4'], p['b4'])
    z4 = _conv2_sc(P, Q, idx.reshape(-1))

    batch_pad = batch_pad_1d.reshape(N_PAD, 1)
    bt = batch_pad.reshape(N_ROW_TILES, ROW_TILE)
    lo = jnp.min(bt, axis=1).astype(jnp.int32)
    hi = jnp.max(bt, axis=1).astype(jnp.int32)
    z5max = _stage3_pool(x1_pad, z4, batch_pad, lo, hi, p['W5'], p['b5'],
                         scales['s4'], p['be4'])
    return _head(z5max, p, scales)


# speedup vs baseline: 3.3424x; 1.0011x over previous
import jax
import jax.numpy as jnp
import numpy as np
from jax import lax
from jax.experimental import pallas as pl
from jax.experimental.pallas import tpu as pltpu
import jax.experimental.pallas.tpu_sc as plsc

N_NODES = 10000
N_GRAPHS = 8
K = 5
EPS = 1e-5

N_PAD = 10240
ROW_TILE = 256
N_ROW_TILES = N_PAD // ROW_TILE


def _stage3_body(lo_ref, hi_ref, x1_ref, z4_ref, batch_ref, w5a_ref, w5b_ref,
                 b5_ref, s4_ref, be4_ref, out_ref):
    i = pl.program_id(0)

    @pl.when(i == 0)
    def _init():
        out_ref[...] = jnp.full_like(out_ref, -jnp.inf)

    x2 = jnp.maximum(z4_ref[...], 0.0) * s4_ref[...] + be4_ref[...]
    z = jax.lax.dot_general(x1_ref[...], w5a_ref[...], (((1,), (1,)), ((), ())),
                            preferred_element_type=jnp.float32)
    z = z + jax.lax.dot_general(x2, w5b_ref[...], (((1,), (1,)), ((), ())),
                                preferred_element_type=jnp.float32)
    z = z + b5_ref[...]
    lo = lo_ref[i]
    hi = hi_ref[i]
    batch = batch_ref[...]

    def body(g, _):
        mask = batch == g
        zm = jnp.max(jnp.where(mask, z, -jnp.inf), axis=0, keepdims=True)
        cur = out_ref[pl.ds(g, 1), :]
        out_ref[pl.ds(g, 1), :] = jnp.maximum(cur, zm)
        return 0

    jax.lax.fori_loop(lo, hi + 1, body, 0)


def _stage3_pool(x1_pad, z4, batch_pad, lo, hi, W5, b5, s4, be4):
    grid_spec = pltpu.PrefetchScalarGridSpec(
        num_scalar_prefetch=2,
        grid=(N_ROW_TILES,),
        in_specs=[
            pl.BlockSpec((ROW_TILE, 64), lambda i, lo, hi: (i, 0)),
            pl.BlockSpec((ROW_TILE, 128), lambda i, lo, hi: (i, 0)),
            pl.BlockSpec((ROW_TILE, 1), lambda i, lo, hi: (i, 0)),
            pl.BlockSpec((1024, 64), lambda i, lo, hi: (0, 0)),
            pl.BlockSpec((1024, 128), lambda i, lo, hi: (0, 0)),
            pl.BlockSpec((1, 1024), lambda i, lo, hi: (0, 0)),
            pl.BlockSpec((1, 128), lambda i, lo, hi: (0, 0)),
            pl.BlockSpec((1, 128), lambda i, lo, hi: (0, 0)),
        ],
        out_specs=pl.BlockSpec((16, 1024), lambda i, lo, hi: (0, 0)),
    )
    out = pl.pallas_call(
        _stage3_body,
        grid_spec=grid_spec,
        out_shape=jax.ShapeDtypeStruct((16, 1024), jnp.float32),
    )(lo, hi, x1_pad, z4, batch_pad, W5[:, :64], W5[:, 64:],
      b5.reshape(1, 1024), s4.reshape(1, 128), be4.reshape(1, 128))
    return out[:N_GRAPHS]


def _head_body(z_ref, s5_ref, be5_ref, w6_ref, b6_ref, s6_ref, be6_ref,
               w7_ref, b7_ref, s7_ref, be7_ref, w8_ref, b8_ref, out_ref):
    x = jnp.maximum(z_ref[...], 0.0) * s5_ref[...] + be5_ref[...]
    z = jax.lax.dot_general(x, w6_ref[...], (((1,), (1,)), ((), ())),
                            preferred_element_type=jnp.float32) + b6_ref[...]
    x = jnp.maximum(z, 0.0) * s6_ref[...] + be6_ref[...]
    z = jax.lax.dot_general(x, w7_ref[...], (((1,), (1,)), ((), ())),
                            preferred_element_type=jnp.float32) + b7_ref[...]
    x = jnp.maximum(z, 0.0) * s7_ref[...] + be7_ref[...]
    out_ref[...] = jax.lax.dot_general(x, w8_ref[...], (((1,), (1,)), ((), ())),
                                       preferred_element_type=jnp.float32) \
        + b8_ref[...]


def _head(z5max, p, s):
    args = (z5max, s['s5'].reshape(1, -1), p['be5'].reshape(1, -1),
            p['W6'], p['b6'].reshape(1, -1), s['s6'].reshape(1, -1),
            p['be6'].reshape(1, -1),
            p['W7'], p['b7'].reshape(1, -1), s['s7'].reshape(1, -1),
            p['be7'].reshape(1, -1),
            p['W8'], p['b8'].reshape(1, -1))
    return pl.pallas_call(
        _head_body,
        out_shape=jax.ShapeDtypeStruct((N_GRAPHS, 40), jnp.float32),
    )(*args)


E_REAL = N_NODES + 160000
E_PAD = 172032
NW = 32
PER_W_E = E_PAD // NW
GC = 1344
NPW = N_PAD // NW

_SC_MESH = dict(core_axis_name="c", subcore_axis_name="s",
                num_cores=2, num_subcores=16)


def _ab_body(pos_ref, wa_ref, wb_ref, b1_ref, a_ref, b_ref):
    a_ref[...] = jax.lax.dot_general(
        pos_ref[...], wa_ref[...], (((1,), (0,)), ((), ())),
        preferred_element_type=jnp.float32) + b1_ref[...]
    b_ref[...] = jax.lax.dot_general(
        pos_ref[...], wb_ref[...], (((1,), (0,)), ((), ())),
        preferred_element_type=jnp.float32)


def _ab_kernel(pos_pad, wa8, wb8, b1):
    return pl.pallas_call(
        _ab_body,
        grid=(N_PAD // 2048,),
        in_specs=[pl.BlockSpec((2048, 8), lambda i: (i, 0)),
                  pl.BlockSpec((8, 64), lambda i: (0, 0)),
                  pl.BlockSpec((8, 64), lambda i: (0, 0)),
                  pl.BlockSpec((1, 64), lambda i: (0, 0))],
        out_specs=[pl.BlockSpec((2048, 64), lambda i: (i, 0)),
                   pl.BlockSpec((2048, 64), lambda i: (i, 0))],
        out_shape=[jax.ShapeDtypeStruct((N_PAD, 64), jnp.float32),
                   jax.ShapeDtypeStruct((N_PAD, 64), jnp.float32)],
    )(pos_pad, wa8, wb8, b1.reshape(1, 64))


def _edge_gather_body(a_hbm, b_hbm, src_hbm, dst_hbm, e1_hbm,
                      sidx, didx, rows, sem):
    wid = lax.axis_index("s") * 2 + lax.axis_index("c")
    base = wid * PER_W_E

    def chunk(ci, _):
        off = base + ci * GC
        pltpu.sync_copy(dst_hbm.at[pl.ds(off, GC)], didx)
        pltpu.sync_copy(src_hbm.at[pl.ds(off, GC)], sidx)
        pltpu.async_copy(a_hbm.at[didx], rows, sem).wait()
        pltpu.async_copy(b_hbm.at[sidx], rows, sem, add=True).wait()
        pltpu.sync_copy(rows, e1_hbm.at[pl.ds(off, GC)])
        return 0

    lax.fori_loop(0, PER_W_E // GC, chunk, 0)


def _edge_gather(A, B, src_pad, dst_pad):
    fn = pl.kernel(
        _edge_gather_body,
        out_type=jax.ShapeDtypeStruct((E_PAD, 64), jnp.float32),
        mesh=plsc.VectorSubcoreMesh(**_SC_MESH),
        scratch_types=[pltpu.VMEM((GC,), jnp.int32),
                       pltpu.VMEM((GC,), jnp.int32),
                       pltpu.VMEM((GC, 64), jnp.float32),
                       pltpu.SemaphoreType.DMA],
        compiler_params=pltpu.CompilerParams(use_tc_tiling_on_sc=False),
    )
    return fn(A, B, src_pad, dst_pad)


def _edge_mlp_body(e1_ref, w2_ref, b2_ref, w3_ref, b3_ref, sc_ref, m_ref):
    i = pl.program_id(0)
    s1 = sc_ref[0:1, :]
    be1 = sc_ref[1:2, :]
    s2 = sc_ref[2:3, :]
    be2 = sc_ref[3:4, :]
    s3 = sc_ref[4:5, :]
    be3 = sc_ref[5:6, :]
    h = jnp.maximum(e1_ref[...], 0.0) * s1 + be1
    z = jax.lax.dot_general(h, w2_ref[...], (((1,), (1,)), ((), ())),
                            preferred_element_type=jnp.float32) + b2_ref[...]
    h = jnp.maximum(z, 0.0) * s2 + be2
    z = jax.lax.dot_general(h, w3_ref[...], (((1,), (1,)), ((), ())),
                            preferred_element_type=jnp.float32) + b3_ref[...]
    h = jnp.maximum(z, 0.0) * s3 + be3
    gid = i * 512 + jax.lax.broadcasted_iota(jnp.int32, (512, 1), 0)
    m_ref[...] = jnp.where(gid < E_REAL, h, -jnp.inf)


def _edge_mlp(E1, p, s):
    sc = jnp.stack([s['s1'], p['be1'], s['s2'], p['be2'], s['s3'], p['be3']])
    return pl.pallas_call(
        _edge_mlp_body,
        grid=(E_PAD // 512,),
        in_specs=[pl.BlockSpec((512, 64), lambda i: (i, 0)),
                  pl.BlockSpec((64, 64), lambda i: (0, 0)),
                  pl.BlockSpec((1, 64), lambda i: (0, 0)),
                  pl.BlockSpec((64, 64), lambda i: (0, 0)),
                  pl.BlockSpec((1, 64), lambda i: (0, 0)),
                  pl.BlockSpec((6, 64), lambda i: (0, 0))],
        out_specs=pl.BlockSpec((512, 64), lambda i: (i, 0)),
        out_shape=jax.ShapeDtypeStruct((E_PAD, 64), jnp.float32),
    )(E1, p['W2'], p['b2'].reshape(1, 64), p['W3'], p['b3'].reshape(1, 64), sc)


KNN_RT = 128
KNN_CT = 512
KNN_NI = N_PAD // KNN_RT
KNN_NJ = N_PAD // KNN_CT
IMAX = np.int32(0x7FFFFFFF)


def _knn_body(rlo, rhi, clo, chi, xall_ref, batchr_ref, batchc_ref,
              w4d_ref, w4b_ref, b4_ref, out_ref, p_ref, q_ref, td, ti):
    i = pl.program_id(0)
    j = pl.program_id(1)

    @pl.when(j == 0)
    def _init():
        td[...] = jnp.full_like(td, jnp.inf)
        ti[...] = jnp.full_like(ti, IMAX)
        xr0 = xall_ref[pl.ds(i * KNN_RT, KNN_RT), :]
        p_ref[...] = jax.lax.dot_general(
            xr0, w4d_ref[...], (((1,), (1,)), ((), ())),
            preferred_element_type=jnp.float32) + b4_ref[...]
        q_ref[...] = jax.lax.dot_general(
            xr0, w4b_ref[...], (((1,), (1,)), ((), ())),
            preferred_element_type=jnp.float32)

    active = jnp.logical_and(rhi[i] >= clo[j], rlo[i] <= chi[j])

    @pl.when(active)
    def _merge():
        xr = xall_ref[pl.ds(i * KNN_RT, KNN_RT), :]
        xc = xall_ref[pl.ds(j * KNN_CT, KNN_CT), :]
        r = jax.lax.dot_general(xr, xc, (((1,), (1,)), ((), ())),
                                preferred_element_type=jnp.float32)
        sqc = jax.lax.dot_general(jnp.ones((1, 64), jnp.float32), xc * xc,
                                  (((1,), (1,)), ((), ())),
                                  preferred_element_type=jnp.float32)
        r = sqc - 2.0 * r
        bm = batchr_ref[...] != batchc_ref[...]
        r = jnp.where(bm, jnp.inf, r)
        colid = jax.lax.broadcasted_iota(jnp.int32, (KNN_RT, KNN_CT), 1) \
            + j * KNN_CT
        cd = jnp.concatenate([td[...], r], axis=1)
        ci = jnp.concatenate([ti[...], colid], axis=1)
        for k in range(K):
            m = jnp.min(cd, axis=1, keepdims=True)
            sel = jnp.min(jnp.where(cd == m, ci, IMAX), axis=1, keepdims=True)
            td[:, k:k + 1] = m
            ti[:, k:k + 1] = sel
            cd = jnp.where(ci == sel, jnp.inf, cd)

    @pl.when(j == KNN_NJ - 1)
    def _emit():
        out_ref[...] = ti[:, :8]


def _knn_topk(x1_pad, batch_pad, W4, b4):
    bt_r = batch_pad.reshape(KNN_NI, KNN_RT)
    bt_c = batch_pad.reshape(KNN_NJ, KNN_CT)
    rlo = bt_r[:, 0].astype(jnp.int32)
    rhi = bt_r[:, -1].astype(jnp.int32)
    clo = bt_c[:, 0].astype(jnp.int32)
    chi = bt_c[:, -1].astype(jnp.int32)
    w4d = W4[:, :64] - W4[:, 64:]
    w4b = W4[:, 64:]
    grid_spec = pltpu.PrefetchScalarGridSpec(
        num_scalar_prefetch=4,
        grid=(KNN_NI, KNN_NJ),
        in_specs=[
            pl.BlockSpec((N_PAD, 64), lambda i, j, *_: (0, 0)),
            pl.BlockSpec((KNN_RT, 1), lambda i, j, *_: (i, 0)),
            pl.BlockSpec((1, KNN_CT), lambda i, j, *_: (0, j)),
            pl.BlockSpec((128, 64), lambda i, j, *_: (0, 0)),
            pl.BlockSpec((128, 64), lambda i, j, *_: (0, 0)),
            pl.BlockSpec((1, 128), lambda i, j, *_: (0, 0)),
        ],
        out_specs=[pl.BlockSpec((KNN_RT, 8), lambda i, j, *_: (i, 0)),
                   pl.BlockSpec((KNN_RT, 128), lambda i, j, *_: (i, 0)),
                   pl.BlockSpec((KNN_RT, 128), lambda i, j, *_: (i, 0))],
        scratch_shapes=[pltpu.VMEM((KNN_RT, 128), jnp.float32),
                        pltpu.VMEM((KNN_RT, 128), jnp.int32)],
    )
    idx8, P, Q = pl.pallas_call(
        _knn_body,
        grid_spec=grid_spec,
        out_shape=[jax.ShapeDtypeStruct((N_PAD, 8), jnp.int32),
                   jax.ShapeDtypeStruct((N_PAD, 128), jnp.float32),
                   jax.ShapeDtypeStruct((N_PAD, 128), jnp.float32)],
        compiler_params=pltpu.CompilerParams(
            dimension_semantics=("arbitrary", "arbitrary")),
    )(rlo, rhi, clo, chi, x1_pad,
      batch_pad.reshape(N_PAD, 1), batch_pad.reshape(1, N_PAD),
      w4d, w4b, b4.reshape(1, 128))
    return idx8[:, :K], P, Q


C2_CH = 64
C2_NCH = NPW // C2_CH


def _conv2_body(p_hbm, q_hbm, idx_hbm, z4_hbm, pvm, qvm, ivm, zvm, sem):
    wid = lax.axis_index("s") * 2 + lax.axis_index("c")
    lo = wid * NPW

    def chunk(c, _):
        nb = lo + c * C2_CH
        pltpu.sync_copy(p_hbm.at[pl.ds(nb * 128, C2_CH * 128)], pvm)
        pltpu.sync_copy(idx_hbm.at[pl.ds(nb * K, C2_CH * K)], ivm)
        pltpu.async_copy(q_hbm.at[ivm], qvm, sem).wait()

        def node(nn, _):
            rowp = jnp.full((16,), 0, jnp.int32) + nn
            for q in range(8):
                col = lax.iota(jnp.int32, 16) + q * 16
                qacc = plsc.load_gather(qvm, [rowp * K, col])
                for k in range(1, K):
                    rowq = rowp * K + k
                    qacc = jnp.maximum(qacc, plsc.load_gather(qvm,
                                                              [rowq, col]))
                acc = plsc.load_gather(pvm, [rowp * 128 + col]) + qacc
                plsc.store_scatter(zvm, [rowp * 128 + col], acc)
            return 0

        lax.fori_loop(0, C2_CH, node, 0)
        pltpu.sync_copy(zvm, z4_hbm.at[pl.ds(nb * 128, C2_CH * 128)])
        return 0

    lax.fori_loop(0, C2_NCH, chunk, 0)


def _conv2_sc(P, Q, idx_flat):
    fn = pl.kernel(
        _conv2_body,
        out_type=jax.ShapeDtypeStruct((N_PAD * 128,), jnp.float32),
        mesh=plsc.VectorSubcoreMesh(**_SC_MESH),
        scratch_types=[pltpu.VMEM((C2_CH * 128,), jnp.float32),
                       pltpu.VMEM((C2_CH * K, 128), jnp.float32),
                       pltpu.VMEM((C2_CH * K,), jnp.int32),
                       pltpu.VMEM((C2_CH * 128,), jnp.float32),
                       pltpu.SemaphoreType.DMA],
        compiler_params=pltpu.CompilerParams(use_tc_tiling_on_sc=False,
                                             needs_layout_passes=False),
    )
    return fn(P.reshape(-1), Q, idx_flat).reshape(N_PAD, 128)


def kernel(pos, batch, edge_index, params):
    p = params
    n = pos.shape[0]
    scales = {k: p['g' + k[1]] / jnp.sqrt(1.0 + EPS)
              for k in ('s1', 's2', 's3', 's4', 's5', 's6', 's7')}

    loops = jnp.arange(n, dtype=edge_index.dtype)
    src = jnp.concatenate([edge_index[0], loops]).astype(jnp.int32)
    dst = jnp.concatenate([edge_index[1], loops]).astype(jnp.int32)
    src_pad = jnp.pad(src, (0, E_PAD - E_REAL))
    dst_pad = jnp.pad(dst, (0, E_PAD - E_REAL), constant_values=N_PAD - 1)
    pos_pad = jnp.pad(pos, ((0, N_PAD - n), (0, 5)))
    W1 = p['W1']
    wa8 = jnp.pad((W1[:, :3] - W1[:, 3:]).T, ((0, 5), (0, 0)))
    wb8 = jnp.pad(W1[:, 3:].T, ((0, 5), (0, 0)))
    A, B = _ab_kernel(pos_pad, wa8, wb8, p['b1'])
    E1 = _edge_gather(A, B, src_pad, dst_pad)
    M = _edge_mlp(E1, p, scales)
    x1 = jax.ops.segment_max(M[:E_REAL], dst, num_segments=n)
    x1_pad_sc = jnp.pad(x1, ((0, N_PAD - n), (0, 0)))

    x1_pad = x1_pad_sc
    batch_pad_1d = jnp.pad(batch.astype(jnp.int32), (0, N_PAD - n),
                           constant_values=N_GRAPHS)
    idx, P, Q = _knn_topk(x1_pad, batch_pad_1d, p['W4'], p['b4'])
    z4 = _conv2_sc(P, Q, idx.reshape(-1))

    batch_pad = batch_pad_1d.reshape(N_PAD, 1)
    bt = batch_pad.reshape(N_ROW_TILES, ROW_TILE)
    lo = jnp.min(bt, axis=1).astype(jnp.int32)
    hi = jnp.max(bt, axis=1).astype(jnp.int32)
    z5max = _stage3_pool(x1_pad, z4, batch_pad, lo, hi, p['W5'], p['b5'],
                         scales['s4'], p['be4'])
    return _head(z5max, p, scales)
